# baseline (speedup 1.0000x reference)
import functools

import jax
import jax.numpy as jnp
from jax import lax
from jax.experimental import pallas as pl
from jax.experimental.pallas import tpu as pltpu
from jax.experimental.pallas import tpu_sc as plsc

B = 8
P = 2048
Q = 2048
NW = 32
NSCB = B
WPB = NW // NSCB
QPW = P // WPB
NQV = QPW // 16
NTV = Q // 16
BIG = 1e30

_mesh = plsc.VectorSubcoreMesh(core_axis_name="c", subcore_axis_name="s")


@functools.partial(
    pl.kernel,
    out_type=(
        jax.ShapeDtypeStruct((NW, 16), jnp.float32),
        jax.ShapeDtypeStruct((NW, Q), jnp.float32),
    ),
    mesh=_mesh,
    scratch_types=[
        pltpu.VMEM((3, QPW), jnp.float32),
        pltpu.VMEM((3, Q), jnp.float32),
        pltpu.VMEM((Q,), jnp.float32),
        pltpu.VMEM((16,), jnp.float32),
    ],
)
def _sc_chamfer(xt_hbm, yt_hbm, minxs_hbm, miny_hbm, xq_v, yt_v, miny_v, sx_v):
    c = lax.axis_index("c")
    s = lax.axis_index("s")
    wid = s * 2 + c
    b = wid // WPB
    qbase = (wid % WPB) * QPW

    pltpu.sync_copy(xt_hbm.at[b, :, pl.ds(qbase, QPW)], xq_v)
    pltpu.sync_copy(yt_hbm.at[b], yt_v)

    def init_j(j, carry):
        miny_v[pl.ds(j * 16, 16)] = jnp.full((16,), BIG, jnp.float32)
        return carry

    lax.fori_loop(0, NTV, init_j, 0)

    iota16 = lax.iota(jnp.int32, 16)
    rot1 = (iota16 + 1) & 15
    unrot = [(iota16 + (16 - r)) & 15 for r in range(16)]

    def qv_loop(i, sumvec):
        qx0 = xq_v[0, pl.ds(i * 16, 16)]
        qy0 = xq_v[1, pl.ds(i * 16, 16)]
        qz0 = xq_v[2, pl.ds(i * 16, 16)]

        def t_loop(j, macc):
            tx = yt_v[0, pl.ds(j * 16, 16)]
            ty = yt_v[1, pl.ds(j * 16, 16)]
            tz = yt_v[2, pl.ds(j * 16, 16)]
            mv = miny_v[pl.ds(j * 16, 16)]
            qx, qy, qz = qx0, qy0, qz0
            out = []
            for r in range(16):
                if r > 0:
                    qx = jnp.take_along_axis(qx, rot1, axis=0)
                    qy = jnp.take_along_axis(qy, rot1, axis=0)
                    qz = jnp.take_along_axis(qz, rot1, axis=0)
                dx = qx - tx
                dy = qy - ty
                dz = qz - tz
                d = dx * dx + dy * dy + dz * dz
                out.append(jnp.minimum(macc[r], d))
                mv = jnp.minimum(mv, d)
            miny_v[pl.ds(j * 16, 16)] = mv
            return tuple(out)

        macc0 = tuple(jnp.full((16,), BIG, jnp.float32) for _ in range(16))
        macc = lax.fori_loop(0, NTV, t_loop, macc0)

        res = macc[0]
        for r in range(1, 16):
            res = jnp.minimum(res, jnp.take_along_axis(macc[r], unrot[r], axis=0))
        return sumvec + jnp.maximum(res, 0.0)

    sumvec = lax.fori_loop(0, NQV, qv_loop, jnp.zeros((16,), jnp.float32))
    sx_v[...] = sumvec
    pltpu.sync_copy(sx_v, minxs_hbm.at[wid])
    pltpu.sync_copy(miny_v, miny_hbm.at[wid])


def _combine_body(minxs_ref, miny_ref, out_ref):
    m = jnp.min(miny_ref[...].reshape(NSCB, WPB, Q), axis=1)
    s = jnp.sum(jnp.maximum(m, 0.0)) + jnp.sum(minxs_ref[...])
    out_ref[0, 0] = s / (float(P) * float(B))


def kernel(x_hat, pos):
    xt = jnp.transpose(x_hat, (0, 2, 1))
    yt = jnp.transpose(pos, (0, 2, 1))

    minxs, miny = _sc_chamfer(xt, yt)

    out = pl.pallas_call(
        _combine_body,
        out_specs=pl.BlockSpec(memory_space=pltpu.SMEM),
        out_shape=jax.ShapeDtypeStruct((1, 1), jnp.float32),
    )(minxs, miny)
    return out[0, 0]


# speedup vs baseline: 4.0338x; 4.0338x over previous
import functools

import jax
import jax.numpy as jnp
from jax import lax
from jax.experimental import pallas as pl
from jax.experimental.pallas import tpu as pltpu
from jax.experimental.pallas import tpu_sc as plsc

B = 8
P = 2048
Q = 2048
DPAD = 8
LANE = 128
NBLK = Q // LANE

NW = 32
NSCB = 1
NTCB = B - NSCB
WPB = NW // NSCB
QPW = P // WPB
NQV = QPW // 16
NTV = Q // 16
BIG = 1e30

_mesh = plsc.VectorSubcoreMesh(core_axis_name="c", subcore_axis_name="s")


@functools.partial(
    pl.kernel,
    out_type=(
        jax.ShapeDtypeStruct((NW, 16), jnp.float32),
        jax.ShapeDtypeStruct((NW, Q), jnp.float32),
    ),
    mesh=_mesh,
    scratch_types=[
        pltpu.VMEM((3, QPW), jnp.float32),
        pltpu.VMEM((3, Q), jnp.float32),
        pltpu.VMEM((Q,), jnp.float32),
        pltpu.VMEM((16,), jnp.float32),
    ],
)
def _sc_chamfer(xt_hbm, yt_hbm, minxs_hbm, miny_hbm, xq_v, yt_v, miny_v, sx_v):
    c = lax.axis_index("c")
    s = lax.axis_index("s")
    wid = s * 2 + c
    b = wid // WPB
    qbase = (wid % WPB) * QPW

    pltpu.sync_copy(xt_hbm.at[b, 0, pl.ds(qbase, QPW)], xq_v.at[0])
    pltpu.sync_copy(xt_hbm.at[b, 1, pl.ds(qbase, QPW)], xq_v.at[1])
    pltpu.sync_copy(xt_hbm.at[b, 2, pl.ds(qbase, QPW)], xq_v.at[2])
    pltpu.sync_copy(yt_hbm.at[b], yt_v)

    def init_j(j, carry):
        miny_v[pl.ds(j * 16, 16)] = jnp.full((16,), BIG, jnp.float32)
        return carry

    lax.fori_loop(0, NTV, init_j, 0)

    iota16 = lax.iota(jnp.int32, 16)
    rot1 = (iota16 + 1) & 15
    unrot = [(iota16 + (16 - r)) & 15 for r in range(16)]

    def qv_loop(i, sumvec):
        qx0 = xq_v[0, pl.ds(i * 16, 16)]
        qy0 = xq_v[1, pl.ds(i * 16, 16)]
        qz0 = xq_v[2, pl.ds(i * 16, 16)]

        def t_loop(j, macc):
            tx = yt_v[0, pl.ds(j * 16, 16)]
            ty = yt_v[1, pl.ds(j * 16, 16)]
            tz = yt_v[2, pl.ds(j * 16, 16)]
            mv = miny_v[pl.ds(j * 16, 16)]
            qx, qy, qz = qx0, qy0, qz0
            out = []
            for r in range(16):
                if r > 0:
                    qx = jnp.take_along_axis(qx, rot1, axis=0)
                    qy = jnp.take_along_axis(qy, rot1, axis=0)
                    qz = jnp.take_along_axis(qz, rot1, axis=0)
                dx = qx - tx
                dy = qy - ty
                dz = qz - tz
                d = dx * dx + dy * dy + dz * dz
                out.append(jnp.minimum(macc[r], d))
                mv = jnp.minimum(mv, d)
            miny_v[pl.ds(j * 16, 16)] = mv
            return tuple(out)

        macc0 = tuple(jnp.full((16,), BIG, jnp.float32) for _ in range(16))
        macc = lax.fori_loop(0, NTV, t_loop, macc0)

        res = macc[0]
        for r in range(1, 16):
            res = jnp.minimum(res, jnp.take_along_axis(macc[r], unrot[r], axis=0))
        return sumvec + jnp.maximum(res, 0.0)

    sumvec = lax.fori_loop(0, NQV, qv_loop, jnp.zeros((16,), jnp.float32))
    sx_v[...] = sumvec
    pltpu.sync_copy(sx_v, minxs_hbm.at[wid])
    pltpu.sync_copy(miny_v, miny_hbm.at[wid])


def _tc_body(x_ref, yt_ref, out_ref, acc_ref):
    b = pl.program_id(0)

    x = x_ref[0]
    yt = yt_ref[0]

    x2 = jnp.sum(x * x, axis=1, keepdims=True)
    aug_x = jnp.concatenate(
        [x, x2, jnp.ones((P, 1), jnp.float32),
         jnp.zeros((P, DPAD - 5), jnp.float32)], axis=1)

    y2 = jnp.sum(yt * yt, axis=0, keepdims=True)
    aug_y = jnp.concatenate(
        [-2.0 * yt, jnp.ones((1, Q), jnp.float32), y2,
         jnp.zeros((DPAD - 5, Q), jnp.float32)], axis=0)

    d = jax.lax.dot_general(
        aug_x, aug_y, (((1,), (0,)), ((), ())),
        preferred_element_type=jnp.float32,
    )

    s = 0.0
    a = None
    for k in range(NBLK):
        dblk = d[:, k * LANE:(k + 1) * LANE]
        a = dblk if a is None else jnp.minimum(a, dblk)
        my = jnp.min(dblk, axis=0)
        s += jnp.sum(jnp.maximum(my, 0.0))

    mx = jnp.min(a, axis=1)
    s += jnp.sum(jnp.maximum(mx, 0.0))

    @pl.when(b == 0)
    def _():
        acc_ref[0, 0] = 0.0

    acc_ref[0, 0] += s

    @pl.when(b == NTCB - 1)
    def _():
        out_ref[0, 0] = acc_ref[0, 0]


def _combine_body(tc_ref, minxs_ref, miny_ref, out_ref):
    m = jnp.min(miny_ref[...], axis=0)
    s = jnp.sum(jnp.maximum(m, 0.0)) + jnp.sum(minxs_ref[...]) + tc_ref[0, 0]
    out_ref[0, 0] = s / (float(P) * float(B))


def kernel(x_hat, pos):
    yt = jnp.transpose(pos, (0, 2, 1))
    xt_sc = jnp.transpose(x_hat[NTCB:], (0, 2, 1))

    minxs, miny = _sc_chamfer(xt_sc, yt[NTCB:])

    tc_sum = pl.pallas_call(
        _tc_body,
        grid=(NTCB,),
        in_specs=[
            pl.BlockSpec((1, P, 3), lambda b: (b, 0, 0)),
            pl.BlockSpec((1, 3, Q), lambda b: (b, 0, 0)),
        ],
        out_specs=pl.BlockSpec(
            (1, 1), lambda b: (0, 0), memory_space=pltpu.SMEM
        ),
        out_shape=jax.ShapeDtypeStruct((1, 1), jnp.float32),
        scratch_shapes=[
            pltpu.SMEM((1, 1), jnp.float32),
        ],
    )(x_hat[:NTCB], yt[:NTCB])

    out = pl.pallas_call(
        _combine_body,
        in_specs=[
            pl.BlockSpec(memory_space=pltpu.SMEM),
            pl.BlockSpec(memory_space=pltpu.VMEM),
            pl.BlockSpec(memory_space=pltpu.VMEM),
        ],
        out_specs=pl.BlockSpec(memory_space=pltpu.SMEM),
        out_shape=jax.ShapeDtypeStruct((1, 1), jnp.float32),
    )(tc_sum, minxs, miny)
    return out[0, 0]


# speedup vs baseline: 7.1964x; 1.7840x over previous
import jax
import jax.numpy as jnp
from jax.experimental import pallas as pl
from jax.experimental.pallas import tpu as pltpu

B = 8
P = 2048
Q = 2048
DPAD = 8
LANE = 128
NBLK = Q // LANE


def _chamfer_body(x_ref, yt_ref, out_ref, acc_ref):
    b = pl.program_id(0)

    x = x_ref[0]
    yt = yt_ref[0]

    x2 = jnp.sum(x * x, axis=1, keepdims=True)
    aug_x = jnp.concatenate(
        [x, x2, jnp.ones((P, 1), jnp.float32),
         jnp.zeros((P, DPAD - 5), jnp.float32)], axis=1)

    y2 = jnp.sum(yt * yt, axis=0, keepdims=True)
    aug_y = jnp.concatenate(
        [-2.0 * yt, jnp.ones((1, Q), jnp.float32), y2,
         jnp.zeros((DPAD - 5, Q), jnp.float32)], axis=0)

    d = jax.lax.dot_general(
        aug_x, aug_y, (((1,), (0,)), ((), ())),
        preferred_element_type=jnp.float32,
    )

    s = 0.0
    a = None
    for k in range(NBLK):
        dblk = d[:, k * LANE:(k + 1) * LANE]
        a = dblk if a is None else jnp.minimum(a, dblk)
        my = jnp.min(dblk, axis=0)
        s += jnp.sum(jnp.maximum(my, 0.0))

    mx = jnp.min(a, axis=1)
    s += jnp.sum(jnp.maximum(mx, 0.0))

    @pl.when(b == 0)
    def _():
        acc_ref[0, 0] = 0.0

    acc_ref[0, 0] += s

    @pl.when(b == B - 1)
    def _():
        out_ref[0, 0] = acc_ref[0, 0] / (float(P) * float(B))


def kernel(x_hat, pos):
    ytp = jnp.transpose(pos, (0, 2, 1))

    out = pl.pallas_call(
        _chamfer_body,
        grid=(B,),
        in_specs=[
            pl.BlockSpec((1, P, 3), lambda b: (b, 0, 0)),
            pl.BlockSpec((1, 3, Q), lambda b: (b, 0, 0)),
        ],
        out_specs=pl.BlockSpec(
            (1, 1), lambda b: (0, 0), memory_space=pltpu.SMEM
        ),
        out_shape=jax.ShapeDtypeStruct((1, 1), jnp.float32),
        scratch_shapes=[
            pltpu.SMEM((1, 1), jnp.float32),
        ],
        compiler_params=pltpu.CompilerParams(
            allow_input_fusion=[False, True],
        ),
    )(x_hat, ytp)
    return out[0, 0]
